# baseline (speedup 1.0000x reference)
import functools

import jax
import jax.numpy as jnp
from jax import lax
from jax.experimental import pallas as pl
from jax.experimental.pallas import tpu as pltpu
from jax.experimental.pallas import tpu_sc as plsc

_UNIVERSE_K = 512
_TRADE_K = 128
_MININT = -(2**31)


def _kth_largest_threshold(keys, k):
    def body(i, prefix):
        cand = prefix | (jnp.int32(1) << (30 - i))
        cnt = jnp.sum((keys >= cand).astype(jnp.int32), axis=1, keepdims=True)
        return jnp.where(cnt >= k, cand, prefix)

    prefix = jnp.zeros((keys.shape[0], 1), jnp.int32)
    return jax.lax.fori_loop(0, 31, body, prefix, unroll=True)


def _layer1_body(x_ref, w1_ref, b1_ref, vol_ref, spr_ref, h_ref, uni_ref):
    ratio = vol_ref[...] / (spr_ref[...] + 1e-8)
    rbits = jax.lax.bitcast_convert_type(ratio, jnp.int32)
    t1 = _kth_largest_threshold(rbits, _UNIVERSE_K)
    uni_ref[...] = (rbits >= t1).astype(jnp.int8)

    h = jnp.dot(x_ref[...], w1_ref[...], preferred_element_type=jnp.float32)
    h_ref[...] = jnp.maximum(h + b1_ref[...], 0.0)


def _keys_body(h_ref, uni_ref, w2_ref, b2_ref, ks_ref):
    logits = jnp.dot(h_ref[...], w2_ref[...], preferred_element_type=jnp.float32)
    s = jax.nn.sigmoid(logits + b2_ref[...])
    ls = s - 0.5
    universe = uni_ref[...] != 0
    abskey = jax.lax.bitcast_convert_type(jnp.abs(ls), jnp.int32) + 1
    signed = jnp.where(ls < 0.0, abskey | jnp.int32(_MININT), abskey)
    ks_ref[...] = jnp.where(universe, signed, 0)


_SHIFTS = (23, 15, 7, 0)
_FIELD_MASKS = (0xFF, 0xFF, 0xFF, 0x7F)
_HI_MASKS = (0, 0x7F800000, 0x7FFF8000, 0x7FFFFF80)


def _sc_select_body(ks_hbm, out_hbm, ks_v, out_v, hist_v, n_groups=8):
    nc = 2
    wid = lax.axis_index("s") * nc + lax.axis_index("c")
    n = 2048
    lane = lax.iota(jnp.int32, 16)
    row_base = lane * n
    ones16 = jnp.ones((16,), jnp.int32)
    zero16 = jnp.zeros((16,), jnp.int32)

    for g in range(n_groups):
        grp = wid * n_groups + g
        e0 = grp * 16 * n
        pltpu.sync_copy(ks_hbm.at[pl.ds(e0, 16 * n)], ks_v)

        prefix = zero16
        kk = jnp.full((16,), _TRADE_K, jnp.int32)
        for lvl in range(4):
            shift = _SHIFTS[lvl]
            fmask = jnp.int32(_FIELD_MASKS[lvl])
            himask = jnp.int32(_HI_MASKS[lvl])

            @plsc.parallel_loop(0, 256, unroll=8)
            def _zero(b, hist_v=hist_v):
                hist_v[pl.ds(b * 16, 16)] = zero16

            @plsc.parallel_loop(0, n, unroll=8)
            def _hist(a, ks_v=ks_v, hist_v=hist_v, prefix=prefix,
                      shift=shift, fmask=fmask, himask=himask):
                col = (a + lane) & (n - 1)
                v = plsc.load_gather(ks_v, [row_base + col])
                m = v & jnp.int32(0x7FFFFFFF)
                hb = lax.shift_right_logical(m, shift) & fmask
                inplay = (m & himask) == prefix
                plsc.addupdate_scatter(hist_v, [hb * 16 + lane], ones16,
                                       mask=inplay)

            def sbody(i, carry, hist_v=hist_v):
                acc, bstar, cnt_above, found = carry
                b = 255 - i
                h = hist_v[pl.ds(b * 16, 16)]
                newacc = acc + h
                take = jnp.logical_and(found == 0, newacc >= kk)
                bstar = jnp.where(take, b, bstar)
                cnt_above = jnp.where(take, acc, cnt_above)
                found = jnp.where(take, ones16, found)
                return (newacc, bstar, cnt_above, found)

            _, bstar, cnt_above, _ = lax.fori_loop(
                0, 256, sbody, (zero16, zero16, zero16, zero16), unroll=4)
            kk = kk - cnt_above
            prefix = prefix | lax.shift_left(bstar, shift)

        t2 = prefix

        @plsc.parallel_loop(0, n, unroll=8,
                            carry=jnp.zeros((16,), jnp.float32))
        def accabs(a, acc, ks_v=ks_v, t2=t2):
            col = (a + lane) & (n - 1)
            v = plsc.load_gather(ks_v, [row_base + col])
            m = v & jnp.int32(0x7FFFFFFF)
            sel = m >= t2
            magbits = jnp.where(sel, m - 1, 0)
            mag = jax.lax.bitcast_convert_type(magbits, jnp.float32)
            return acc + mag

        scale = 1.0 / (accabs + 1e-8)

        @plsc.parallel_loop(0, n, unroll=8)
        def _emit(a, ks_v=ks_v, out_v=out_v, t2=t2, scale=scale):
            col = (a + lane) & (n - 1)
            v = plsc.load_gather(ks_v, [row_base + col])
            m = v & jnp.int32(0x7FFFFFFF)
            sel = m >= t2
            magbits = jnp.where(sel, m - 1, 0)
            mag = jax.lax.bitcast_convert_type(magbits, jnp.float32)
            val = jnp.where(v < 0, -mag, mag) * scale
            plsc.store_scatter(out_v, [row_base + col], val)

        pltpu.sync_copy(out_v, out_hbm.at[pl.ds(e0, 16 * n)])


def kernel(signal_features, volatility, spread, W1, b1, W2, b2):
    B, D_IN = signal_features.shape
    _, H = W1.shape
    N = W2.shape[1]

    C = 2
    R = B // C
    BM1 = 256
    BM2 = 128
    b1r = b1.reshape(1, H)
    b2r = b2.reshape(1, N)
    mesh = plsc.VectorSubcoreMesh(core_axis_name="c", subcore_axis_name="s")
    n_groups = R // (32 * 16)
    sc_select = functools.partial(_sc_select_body, n_groups=n_groups)

    actions = []
    for c in range(C):
        rows = slice(c * R, (c + 1) * R)
        h, universe = pl.pallas_call(
            _layer1_body,
            grid=(R // BM1,),
            in_specs=[
                pl.BlockSpec((BM1, D_IN), lambda i: (i, 0)),
                pl.BlockSpec((D_IN, H), lambda i: (0, 0)),
                pl.BlockSpec((1, H), lambda i: (0, 0)),
                pl.BlockSpec((BM1, N), lambda i: (i, 0)),
                pl.BlockSpec((BM1, N), lambda i: (i, 0)),
            ],
            out_specs=[
                pl.BlockSpec((BM1, H), lambda i: (i, 0)),
                pl.BlockSpec((BM1, N), lambda i: (i, 0)),
            ],
            out_shape=[
                jax.ShapeDtypeStruct((R, H), jnp.float32),
                jax.ShapeDtypeStruct((R, N), jnp.int8),
            ],
            compiler_params=pltpu.CompilerParams(
                dimension_semantics=("arbitrary",),
            ),
        )(signal_features[rows], W1, b1r, volatility[rows], spread[rows])

        ks = pl.pallas_call(
            _keys_body,
            grid=(R // BM2,),
            in_specs=[
                pl.BlockSpec((BM2, H), lambda i: (i, 0)),
                pl.BlockSpec((BM2, N), lambda i: (i, 0)),
                pl.BlockSpec((H, N), lambda i: (0, 0)),
                pl.BlockSpec((1, N), lambda i: (0, 0)),
            ],
            out_specs=pl.BlockSpec((BM2, N), lambda i: (i, 0)),
            out_shape=jax.ShapeDtypeStruct((R, N), jnp.int32),
            compiler_params=pltpu.CompilerParams(
                dimension_semantics=("arbitrary",),
            ),
        )(h, universe, W2, b2r)

        action_flat = pl.kernel(
            sc_select,
            mesh=mesh,
            out_type=jax.ShapeDtypeStruct((R * N,), jnp.float32),
            scratch_types=[
                pltpu.VMEM((16 * N,), jnp.int32),
                pltpu.VMEM((16 * N,), jnp.float32),
                pltpu.VMEM((256 * 16,), jnp.int32),
            ],
            compiler_params=pltpu.CompilerParams(needs_layout_passes=False),
        )(ks.reshape(-1))
        actions.append(action_flat.reshape(R, N))

    action = jnp.concatenate(actions, axis=0)
    return (action, jnp.zeros_like(action))


# speedup vs baseline: 1.2135x; 1.2135x over previous
import functools

import jax
import jax.numpy as jnp
from jax import lax
from jax.experimental import pallas as pl
from jax.experimental.pallas import tpu as pltpu
from jax.experimental.pallas import tpu_sc as plsc

_UNIVERSE_K = 512
_TRADE_K = 128
_MININT = -(2**31)


def _kth_largest_threshold(keys, k):
    def body(i, prefix):
        cand = prefix | (jnp.int32(1) << (30 - i))
        cnt = jnp.sum((keys >= cand).astype(jnp.int32), axis=1, keepdims=True)
        return jnp.where(cnt >= k, cand, prefix)

    prefix = jnp.zeros((keys.shape[0], 1), jnp.int32)
    return jax.lax.fori_loop(0, 31, body, prefix, unroll=True)


def _layer1_body(x_ref, w1_ref, b1_ref, vol_ref, spr_ref, h_ref, uni_ref):
    ratio = vol_ref[...] / (spr_ref[...] + 1e-8)
    rbits = jax.lax.bitcast_convert_type(ratio, jnp.int32)
    t1 = _kth_largest_threshold(rbits, _UNIVERSE_K)
    uni_ref[...] = (rbits >= t1).astype(jnp.int8)

    h = jnp.dot(x_ref[...], w1_ref[...], preferred_element_type=jnp.float32)
    h_ref[...] = jnp.maximum(h + b1_ref[...], 0.0)


def _keys_body(h_ref, uni_ref, w2_ref, b2_ref, ks_ref):
    logits = jnp.dot(h_ref[...], w2_ref[...], preferred_element_type=jnp.float32)
    s = jax.nn.sigmoid(logits + b2_ref[...])
    ls = s - 0.5
    universe = uni_ref[...] != 0
    abskey = jax.lax.bitcast_convert_type(jnp.abs(ls), jnp.int32) + 1
    signed = jnp.where(ls < 0.0, abskey | jnp.int32(_MININT), abskey)
    ks_ref[...] = jnp.where(universe, signed, 0)


_SHIFTS = (23, 15, 7, 0)
_FIELD_MASKS = (0xFF, 0xFF, 0xFF, 0x7F)
_HI_MASKS = (0, 0x7F800000, 0x7FFF8000, 0x7FFFFF80)


def _sc_select_body(ks_hbm, out_hbm, ks_a, ks_b, out_v, hist_v,
                    sin_a, sin_b, sout, n_groups=8):
    nc = 2
    wid = lax.axis_index("s") * nc + lax.axis_index("c")
    n = 2048
    lane = lax.iota(jnp.int32, 16)
    row_base = lane * n
    ones16 = jnp.ones((16,), jnp.int32)
    zero16 = jnp.zeros((16,), jnp.int32)

    bufs = (ks_a, ks_b)
    sins = (sin_a, sin_b)
    base0 = wid * n_groups * 16 * n
    pltpu.async_copy(ks_hbm.at[pl.ds(base0, 16 * n)], ks_a, sin_a)

    for g in range(n_groups):
        grp = wid * n_groups + g
        e0 = grp * 16 * n
        ks_v = bufs[g % 2]
        if g + 1 < n_groups:
            pltpu.async_copy(ks_hbm.at[pl.ds(e0 + 16 * n, 16 * n)],
                             bufs[(g + 1) % 2], sins[(g + 1) % 2])
        pltpu.make_async_copy(ks_hbm.at[pl.ds(e0, 16 * n)], ks_v,
                              sins[g % 2]).wait()

        prefix = zero16
        kk = jnp.full((16,), _TRADE_K, jnp.int32)
        for lvl in range(4):
            shift = _SHIFTS[lvl]
            fmask = jnp.int32(_FIELD_MASKS[lvl])
            himask = jnp.int32(_HI_MASKS[lvl])

            @plsc.parallel_loop(0, 256, unroll=8)
            def _zero(b, hist_v=hist_v):
                hist_v[pl.ds(b * 16, 16)] = zero16

            @plsc.parallel_loop(0, n, unroll=8)
            def _hist(a, ks_v=ks_v, hist_v=hist_v, prefix=prefix,
                      shift=shift, fmask=fmask, himask=himask):
                col = (a + lane) & (n - 1)
                v = plsc.load_gather(ks_v, [row_base + col])
                m = v & jnp.int32(0x7FFFFFFF)
                hb = lax.shift_right_logical(m, shift) & fmask
                inplay = (m & himask) == prefix
                plsc.addupdate_scatter(hist_v, [hb * 16 + lane], ones16,
                                       mask=inplay)

            def sbody(i, carry, hist_v=hist_v):
                acc, bstar, cnt_above, found = carry
                b = 255 - i
                h = hist_v[pl.ds(b * 16, 16)]
                newacc = acc + h
                take = jnp.logical_and(found == 0, newacc >= kk)
                bstar = jnp.where(take, b, bstar)
                cnt_above = jnp.where(take, acc, cnt_above)
                found = jnp.where(take, ones16, found)
                return (newacc, bstar, cnt_above, found)

            _, bstar, cnt_above, _ = lax.fori_loop(
                0, 256, sbody, (zero16, zero16, zero16, zero16), unroll=4)
            kk = kk - cnt_above
            prefix = prefix | lax.shift_left(bstar, shift)

        t2 = prefix

        @plsc.parallel_loop(0, n, unroll=8,
                            carry=jnp.zeros((16,), jnp.float32))
        def accabs(a, acc, ks_v=ks_v, t2=t2):
            col = (a + lane) & (n - 1)
            v = plsc.load_gather(ks_v, [row_base + col])
            m = v & jnp.int32(0x7FFFFFFF)
            sel = m >= t2
            magbits = jnp.where(sel, m - 1, 0)
            mag = jax.lax.bitcast_convert_type(magbits, jnp.float32)
            return acc + mag

        scale = 1.0 / (accabs + 1e-8)

        if g > 0:
            pltpu.make_async_copy(
                out_v, out_hbm.at[pl.ds(e0 - 16 * n, 16 * n)], sout).wait()

        @plsc.parallel_loop(0, n, unroll=8)
        def _emit(a, ks_v=ks_v, out_v=out_v, t2=t2, scale=scale):
            col = (a + lane) & (n - 1)
            v = plsc.load_gather(ks_v, [row_base + col])
            m = v & jnp.int32(0x7FFFFFFF)
            sel = m >= t2
            magbits = jnp.where(sel, m - 1, 0)
            mag = jax.lax.bitcast_convert_type(magbits, jnp.float32)
            val = jnp.where(v < 0, -mag, mag) * scale
            plsc.store_scatter(out_v, [row_base + col], val)

        if g + 1 < n_groups:
            pltpu.async_copy(out_v, out_hbm.at[pl.ds(e0, 16 * n)], sout)
        else:
            pltpu.sync_copy(out_v, out_hbm.at[pl.ds(e0, 16 * n)])


def kernel(signal_features, volatility, spread, W1, b1, W2, b2):
    B, D_IN = signal_features.shape
    _, H = W1.shape
    N = W2.shape[1]

    C = 1
    R = B // C
    BM1 = 256
    BM2 = 128
    b1r = b1.reshape(1, H)
    b2r = b2.reshape(1, N)
    mesh = plsc.VectorSubcoreMesh(core_axis_name="c", subcore_axis_name="s")
    n_groups = R // (32 * 16)
    sc_select = functools.partial(_sc_select_body, n_groups=n_groups)

    actions = []
    for c in range(C):
        rows = slice(c * R, (c + 1) * R)
        h, universe = pl.pallas_call(
            _layer1_body,
            grid=(R // BM1,),
            in_specs=[
                pl.BlockSpec((BM1, D_IN), lambda i: (i, 0)),
                pl.BlockSpec((D_IN, H), lambda i: (0, 0)),
                pl.BlockSpec((1, H), lambda i: (0, 0)),
                pl.BlockSpec((BM1, N), lambda i: (i, 0)),
                pl.BlockSpec((BM1, N), lambda i: (i, 0)),
            ],
            out_specs=[
                pl.BlockSpec((BM1, H), lambda i: (i, 0)),
                pl.BlockSpec((BM1, N), lambda i: (i, 0)),
            ],
            out_shape=[
                jax.ShapeDtypeStruct((R, H), jnp.float32),
                jax.ShapeDtypeStruct((R, N), jnp.int8),
            ],
            compiler_params=pltpu.CompilerParams(
                dimension_semantics=("arbitrary",),
            ),
        )(signal_features[rows], W1, b1r, volatility[rows], spread[rows])

        ks = pl.pallas_call(
            _keys_body,
            grid=(R // BM2,),
            in_specs=[
                pl.BlockSpec((BM2, H), lambda i: (i, 0)),
                pl.BlockSpec((BM2, N), lambda i: (i, 0)),
                pl.BlockSpec((H, N), lambda i: (0, 0)),
                pl.BlockSpec((1, N), lambda i: (0, 0)),
            ],
            out_specs=pl.BlockSpec((BM2, N), lambda i: (i, 0)),
            out_shape=jax.ShapeDtypeStruct((R, N), jnp.int32),
            compiler_params=pltpu.CompilerParams(
                dimension_semantics=("arbitrary",),
            ),
        )(h, universe, W2, b2r)

        action_flat = pl.kernel(
            sc_select,
            mesh=mesh,
            out_type=jax.ShapeDtypeStruct((R * N,), jnp.float32),
            scratch_types=[
                pltpu.VMEM((16 * N,), jnp.int32),
                pltpu.VMEM((16 * N,), jnp.int32),
                pltpu.VMEM((16 * N,), jnp.float32),
                pltpu.VMEM((256 * 16,), jnp.int32),
                pltpu.SemaphoreType.DMA,
                pltpu.SemaphoreType.DMA,
                pltpu.SemaphoreType.DMA,
            ],
            compiler_params=pltpu.CompilerParams(needs_layout_passes=False),
        )(ks.reshape(-1))
        actions.append(action_flat.reshape(R, N))

    action = jnp.concatenate(actions, axis=0)
    return (action, jnp.zeros_like(action))


# speedup vs baseline: 1.2481x; 1.0285x over previous
import functools

import jax
import jax.numpy as jnp
from jax import lax
from jax.experimental import pallas as pl
from jax.experimental.pallas import tpu as pltpu
from jax.experimental.pallas import tpu_sc as plsc

_UNIVERSE_K = 512
_TRADE_K = 128
_MININT = -(2**31)


def _kth_largest_threshold(keys, k):
    def body(i, prefix):
        cand = prefix | (jnp.int32(1) << (30 - i))
        cnt = jnp.sum((keys >= cand).astype(jnp.int32), axis=1, keepdims=True)
        return jnp.where(cnt >= k, cand, prefix)

    prefix = jnp.zeros((keys.shape[0], 1), jnp.int32)
    return jax.lax.fori_loop(0, 31, body, prefix, unroll=True)


def _layer1_body(x_ref, w1_ref, b1_ref, vol_ref, spr_ref, h_ref, uni_ref):
    ratio = vol_ref[...] / (spr_ref[...] + 1e-8)
    rbits = jax.lax.bitcast_convert_type(ratio, jnp.int32)
    t1 = _kth_largest_threshold(rbits, _UNIVERSE_K)
    uni_ref[...] = (rbits >= t1).astype(jnp.int8)

    h = jnp.dot(x_ref[...], w1_ref[...], preferred_element_type=jnp.float32)
    h_ref[...] = jnp.maximum(h + b1_ref[...], 0.0)


def _keys_body(h_ref, uni_ref, w2_ref, b2_ref, ks_ref):
    logits = jnp.dot(h_ref[...], w2_ref[...], preferred_element_type=jnp.float32)
    s = jax.nn.sigmoid(logits + b2_ref[...])
    ls = s - 0.5
    universe = uni_ref[...] != 0
    abskey = jax.lax.bitcast_convert_type(jnp.abs(ls), jnp.int32) + 1
    signed = jnp.where(ls < 0.0, abskey | jnp.int32(_MININT), abskey)
    ks_ref[...] = jnp.where(universe, signed, 0)


_SHIFTS = (23, 15, 7, 0)
_FIELD_MASKS = (0xFF, 0xFF, 0xFF, 0x7F)
_HI_MASKS = (0, 0x7F800000, 0x7FFF8000, 0x7FFFFF80)


def _sc_select_body(ks_hbm, out_hbm, ks_a, ks_b, out_v, hist_v,
                    sin_a, sin_b, sout, n_groups=8):
    nc = 2
    wid = lax.axis_index("s") * nc + lax.axis_index("c")
    n = 2048
    lane = lax.iota(jnp.int32, 16)
    row_base = lane * n
    ones16 = jnp.ones((16,), jnp.int32)
    zero16 = jnp.zeros((16,), jnp.int32)

    bufs = (ks_a, ks_b)
    sins = (sin_a, sin_b)
    r0_first = wid * n_groups * 16
    pltpu.async_copy(ks_hbm.at[pl.ds(r0_first, 16)], ks_a, sin_a)

    for g in range(n_groups):
        grp = wid * n_groups + g
        r0 = grp * 16
        ks_v = bufs[g % 2]
        if g + 1 < n_groups:
            pltpu.async_copy(ks_hbm.at[pl.ds(r0 + 16, 16)],
                             bufs[(g + 1) % 2], sins[(g + 1) % 2])
        pltpu.make_async_copy(ks_hbm.at[pl.ds(r0, 16)], ks_v,
                              sins[g % 2]).wait()

        prefix = zero16
        kk = jnp.full((16,), _TRADE_K, jnp.int32)
        for lvl in range(4):
            shift = _SHIFTS[lvl]
            fmask = jnp.int32(_FIELD_MASKS[lvl])
            himask = jnp.int32(_HI_MASKS[lvl])

            @plsc.parallel_loop(0, 256, unroll=8)
            def _zero(b, hist_v=hist_v):
                hist_v[pl.ds(b * 16, 16)] = zero16

            @plsc.parallel_loop(0, n, unroll=8)
            def _hist(a, ks_v=ks_v, hist_v=hist_v, prefix=prefix,
                      shift=shift, fmask=fmask, himask=himask):
                col = (a + lane) & (n - 1)
                v = plsc.load_gather(ks_v, [lane, col])
                m = v & jnp.int32(0x7FFFFFFF)
                hb = lax.shift_right_logical(m, shift) & fmask
                inplay = (m & himask) == prefix
                plsc.addupdate_scatter(hist_v, [hb * 16 + lane], ones16,
                                       mask=inplay)

            def sbody(i, carry, hist_v=hist_v):
                acc, bstar, cnt_above, found = carry
                b = 255 - i
                h = hist_v[pl.ds(b * 16, 16)]
                newacc = acc + h
                take = jnp.logical_and(found == 0, newacc >= kk)
                bstar = jnp.where(take, b, bstar)
                cnt_above = jnp.where(take, acc, cnt_above)
                found = jnp.where(take, ones16, found)
                return (newacc, bstar, cnt_above, found)

            _, bstar, cnt_above, _ = lax.fori_loop(
                0, 256, sbody, (zero16, zero16, zero16, zero16), unroll=4)
            kk = kk - cnt_above
            prefix = prefix | lax.shift_left(bstar, shift)

        t2 = prefix

        @plsc.parallel_loop(0, n, unroll=8,
                            carry=jnp.zeros((16,), jnp.float32))
        def accabs(a, acc, ks_v=ks_v, t2=t2):
            col = (a + lane) & (n - 1)
            v = plsc.load_gather(ks_v, [lane, col])
            m = v & jnp.int32(0x7FFFFFFF)
            sel = m >= t2
            magbits = jnp.where(sel, m - 1, 0)
            mag = jax.lax.bitcast_convert_type(magbits, jnp.float32)
            return acc + mag

        scale = 1.0 / (accabs + 1e-8)

        if g > 0:
            pltpu.make_async_copy(
                out_v, out_hbm.at[pl.ds(r0 - 16, 16)], sout).wait()

        @plsc.parallel_loop(0, n, unroll=8)
        def _emit(a, ks_v=ks_v, out_v=out_v, t2=t2, scale=scale):
            col = (a + lane) & (n - 1)
            v = plsc.load_gather(ks_v, [lane, col])
            m = v & jnp.int32(0x7FFFFFFF)
            sel = m >= t2
            magbits = jnp.where(sel, m - 1, 0)
            mag = jax.lax.bitcast_convert_type(magbits, jnp.float32)
            val = jnp.where(v < 0, -mag, mag) * scale
            plsc.store_scatter(out_v, [lane, col], val)

        if g + 1 < n_groups:
            pltpu.async_copy(out_v, out_hbm.at[pl.ds(r0, 16)], sout)
        else:
            pltpu.sync_copy(out_v, out_hbm.at[pl.ds(r0, 16)])


def kernel(signal_features, volatility, spread, W1, b1, W2, b2):
    B, D_IN = signal_features.shape
    _, H = W1.shape
    N = W2.shape[1]

    C = 1
    R = B // C
    BM1 = 256
    BM2 = 128
    b1r = b1.reshape(1, H)
    b2r = b2.reshape(1, N)
    mesh = plsc.VectorSubcoreMesh(core_axis_name="c", subcore_axis_name="s")
    n_groups = R // (32 * 16)
    sc_select = functools.partial(_sc_select_body, n_groups=n_groups)

    actions = []
    for c in range(C):
        rows = slice(c * R, (c + 1) * R)
        h, universe = pl.pallas_call(
            _layer1_body,
            grid=(R // BM1,),
            in_specs=[
                pl.BlockSpec((BM1, D_IN), lambda i: (i, 0)),
                pl.BlockSpec((D_IN, H), lambda i: (0, 0)),
                pl.BlockSpec((1, H), lambda i: (0, 0)),
                pl.BlockSpec((BM1, N), lambda i: (i, 0)),
                pl.BlockSpec((BM1, N), lambda i: (i, 0)),
            ],
            out_specs=[
                pl.BlockSpec((BM1, H), lambda i: (i, 0)),
                pl.BlockSpec((BM1, N), lambda i: (i, 0)),
            ],
            out_shape=[
                jax.ShapeDtypeStruct((R, H), jnp.float32),
                jax.ShapeDtypeStruct((R, N), jnp.int8),
            ],
            compiler_params=pltpu.CompilerParams(
                dimension_semantics=("arbitrary",),
            ),
        )(signal_features[rows], W1, b1r, volatility[rows], spread[rows])

        ks = pl.pallas_call(
            _keys_body,
            grid=(R // BM2,),
            in_specs=[
                pl.BlockSpec((BM2, H), lambda i: (i, 0)),
                pl.BlockSpec((BM2, N), lambda i: (i, 0)),
                pl.BlockSpec((H, N), lambda i: (0, 0)),
                pl.BlockSpec((1, N), lambda i: (0, 0)),
            ],
            out_specs=pl.BlockSpec((BM2, N), lambda i: (i, 0)),
            out_shape=jax.ShapeDtypeStruct((R, N), jnp.int32),
            compiler_params=pltpu.CompilerParams(
                dimension_semantics=("arbitrary",),
            ),
        )(h, universe, W2, b2r)

        action_flat = pl.kernel(
            sc_select,
            mesh=mesh,
            out_type=jax.ShapeDtypeStruct((R, N), jnp.float32),
            scratch_types=[
                pltpu.VMEM((16, N), jnp.int32),
                pltpu.VMEM((16, N), jnp.int32),
                pltpu.VMEM((16, N), jnp.float32),
                pltpu.VMEM((256 * 16,), jnp.int32),
                pltpu.SemaphoreType.DMA,
                pltpu.SemaphoreType.DMA,
                pltpu.SemaphoreType.DMA,
            ],
            compiler_params=pltpu.CompilerParams(needs_layout_passes=False),
        )(ks)
        actions.append(action_flat)

    action = jnp.concatenate(actions, axis=0)
    return (action, jnp.zeros_like(action))
